# baseline (speedup 1.0000x reference)
import functools

import jax
import jax.numpy as jnp
from jax import lax
from jax.experimental import pallas as pl
from jax.experimental.pallas import tpu as pltpu
from jax.experimental.pallas import tpu_sc as plsc

PAD = 0
EPS = 1e-5
NC = 2
NS = 16
NW = NC * NS
SPLIT = 4
TS = 512
CR = 32


def _make_sc_gather(N, V, D):
    TPW = N // NW
    NCH = max(TPW // CR, 1)
    CRW = min(TPW, CR)

    mesh = plsc.VectorSubcoreMesh(core_axis_name="c", subcore_axis_name="s")

    @functools.partial(
        pl.kernel,
        out_type=jax.ShapeDtypeStruct((N, D), jnp.float32),
        mesh=mesh,
        scratch_types=[
            pltpu.VMEM((TPW,), jnp.int32),
            pltpu.VMEM((CRW, D), jnp.float32),
            pltpu.VMEM((CRW, D), jnp.float32),
            pltpu.SemaphoreType.DMA,
            pltpu.SemaphoreType.DMA,
            pltpu.SemaphoreType.DMA,
            pltpu.SemaphoreType.DMA,
        ],
    )
    def sc_gather(idx_hbm, tab_hbm, out_hbm, idx_v, buf0, buf1,
                  sg0, sg1, so0, so1):
        wid = lax.axis_index("s") * NC + lax.axis_index("c")
        base = wid * TPW
        pltpu.sync_copy(idx_hbm.at[pl.ds(base, TPW)], idx_v)

        bufs = (buf0, buf1)
        sgs = (sg0, sg1)
        sos = (so0, so1)
        gh = {}
        oh = {}
        gh[0] = pltpu.async_copy(
            tab_hbm.at[idx_v.at[pl.ds(0, CRW)]], bufs[0], sgs[0])
        for ci in range(NCH):
            gh[ci].wait()
            if ci + 1 < NCH:
                if ci - 1 >= 0:
                    oh[ci - 1].wait()
                gh[ci + 1] = pltpu.async_copy(
                    tab_hbm.at[idx_v.at[pl.ds((ci + 1) * CRW, CRW)]],
                    bufs[(ci + 1) % 2], sgs[(ci + 1) % 2])
            oh[ci] = pltpu.async_copy(
                bufs[ci % 2], out_hbm.at[pl.ds(base + ci * CRW, CRW)],
                sos[ci % 2])
        for ci in range(max(NCH - 2, 0), NCH):
            oh[ci].wait()

    return sc_gather


def _tc_ln_piece(B, S, D, NSEG, p, seq, lbl, tok_p, pos, seg, gam, bet,
                 carry):
    SP = S // SPLIT
    NSBP = SP // TS
    i0 = p * NSBP

    def body(seq_ref, lbl_ref, tok_ref, pos_ref, seg_ref, gam_ref, bet_ref,
             *rest):
        o_ref = rest[-1]
        b = pl.program_id(1)
        tok = tok_ref[0]
        pos_x = pos_ref[...]
        m = (seq_ref[b] != PAD).astype(jnp.float32)[:, None]
        lbf = lbl_ref[b].astype(jnp.float32)[:, None]
        seg0 = seg_ref[0:1, :]
        dseg = seg_ref[1:2, :] - seg0
        x = (tok + pos_x) * m + seg0 + lbf * dseg
        mean = jnp.mean(x, axis=1, keepdims=True)
        xc = x - mean
        var = jnp.mean(xc * xc, axis=1, keepdims=True)
        o_ref[0] = xc * lax.rsqrt(var + EPS) * gam_ref[...] + bet_ref[...]

    in_specs = [
        pl.BlockSpec((B, TS), lambda i, j: (0, i0 + i)),
        pl.BlockSpec((B, TS), lambda i, j: (0, i0 + i)),
        pl.BlockSpec((1, TS, D), lambda i, j: (j, i, 0)),
        pl.BlockSpec((TS, D), lambda i, j: (i0 + i, 0)),
        pl.BlockSpec((NSEG, D), lambda i, j: (0, 0)),
        pl.BlockSpec((1, D), lambda i, j: (0, 0)),
        pl.BlockSpec((1, D), lambda i, j: (0, 0)),
    ]
    args = [seq, lbl, tok_p, pos, seg, gam, bet]
    aliases = {}
    if carry is not None:
        in_specs.append(pl.BlockSpec(memory_space=pl.ANY))
        args.append(carry)
        aliases = {7: 0}

    return pl.pallas_call(
        body,
        grid=(NSBP, B),
        in_specs=in_specs,
        out_specs=pl.BlockSpec((1, TS, D), lambda i, j: (j, i0 + i, 0)),
        out_shape=jax.ShapeDtypeStruct((B, S, D), jnp.float32),
        input_output_aliases=aliases,
    )(*args)


def kernel(sequence, segment_label, token_table, pos_table, seg_table, gamma,
           beta):
    B, S = sequence.shape
    V, D = token_table.shape
    NSEG = seg_table.shape[0]
    SP = S // SPLIT
    seq = sequence.astype(jnp.int32)
    lbl = segment_label.astype(jnp.int32)
    gam = gamma.reshape(1, D)
    bet = beta.reshape(1, D)
    pos = pos_table[:S]

    sc_gather = _make_sc_gather(B * SP, V, D)
    gathered = [
        sc_gather(lax.slice(seq, (0, p * SP), (B, (p + 1) * SP))
                  .reshape(-1), token_table).reshape(B, SP, D)
        for p in range(SPLIT)
    ]

    out = None
    for p in range(SPLIT):
        out = _tc_ln_piece(B, S, D, NSEG, p, seq, lbl, gathered[p], pos,
                           seg_table, gam, bet, out)
    return out


# speedup vs baseline: 1.0719x; 1.0719x over previous
import functools

import jax
import jax.numpy as jnp
from jax import lax
from jax.experimental import pallas as pl
from jax.experimental.pallas import tpu as pltpu
from jax.experimental.pallas import tpu_sc as plsc

PAD = 0
EPS = 1e-5
NC = 2
NS = 16
NW = NC * NS
SPLIT = 2
TS = 512
CR = 64


def _make_sc_gather(N, V, D):
    TPW = N // NW
    NCH = max(TPW // CR, 1)
    CRW = min(TPW, CR)

    mesh = plsc.VectorSubcoreMesh(core_axis_name="c", subcore_axis_name="s")

    @functools.partial(
        pl.kernel,
        out_type=jax.ShapeDtypeStruct((N, D), jnp.float32),
        mesh=mesh,
        scratch_types=[
            pltpu.VMEM((TPW,), jnp.int32),
            pltpu.VMEM((CRW, D), jnp.float32),
            pltpu.VMEM((CRW, D), jnp.float32),
            pltpu.SemaphoreType.DMA,
            pltpu.SemaphoreType.DMA,
            pltpu.SemaphoreType.DMA,
            pltpu.SemaphoreType.DMA,
        ],
    )
    def sc_gather(idx_hbm, tab_hbm, out_hbm, idx_v, buf0, buf1,
                  sg0, sg1, so0, so1):
        wid = lax.axis_index("s") * NC + lax.axis_index("c")
        base = wid * TPW
        pltpu.sync_copy(idx_hbm.at[pl.ds(base, TPW)], idx_v)

        bufs = (buf0, buf1)
        sgs = (sg0, sg1)
        sos = (so0, so1)
        gh = {}
        oh = {}
        gh[0] = pltpu.async_copy(
            tab_hbm.at[idx_v.at[pl.ds(0, CRW)]], bufs[0], sgs[0])
        for ci in range(NCH):
            gh[ci].wait()
            if ci + 1 < NCH:
                if ci - 1 >= 0:
                    oh[ci - 1].wait()
                gh[ci + 1] = pltpu.async_copy(
                    tab_hbm.at[idx_v.at[pl.ds((ci + 1) * CRW, CRW)]],
                    bufs[(ci + 1) % 2], sgs[(ci + 1) % 2])
            oh[ci] = pltpu.async_copy(
                bufs[ci % 2], out_hbm.at[pl.ds(base + ci * CRW, CRW)],
                sos[ci % 2])
        for ci in range(max(NCH - 2, 0), NCH):
            oh[ci].wait()

    return sc_gather


def _tc_ln_piece(B, S, D, NSEG, p, seq, lbl, tok_p, pos, seg, gam, bet,
                 carry):
    SP = S // SPLIT
    NSBP = SP // TS
    i0 = p * NSBP

    def body(seq_ref, lbl_ref, tok_ref, pos_ref, seg_ref, gam_ref, bet_ref,
             *rest):
        o_ref = rest[-1]
        b = pl.program_id(1)
        tok = tok_ref[0]
        pos_x = pos_ref[...]
        m = (seq_ref[b] != PAD).astype(jnp.float32)[:, None]
        lbf = lbl_ref[b].astype(jnp.float32)[:, None]
        seg0 = seg_ref[0:1, :]
        dseg = seg_ref[1:2, :] - seg0
        x = (tok + pos_x) * m + seg0 + lbf * dseg
        mean = jnp.mean(x, axis=1, keepdims=True)
        xc = x - mean
        var = jnp.mean(xc * xc, axis=1, keepdims=True)
        o_ref[0] = xc * lax.rsqrt(var + EPS) * gam_ref[...] + bet_ref[...]

    in_specs = [
        pl.BlockSpec((B, TS), lambda i, j: (0, i0 + i)),
        pl.BlockSpec((B, TS), lambda i, j: (0, i0 + i)),
        pl.BlockSpec((1, TS, D), lambda i, j: (j, i, 0)),
        pl.BlockSpec((TS, D), lambda i, j: (i0 + i, 0)),
        pl.BlockSpec((NSEG, D), lambda i, j: (0, 0)),
        pl.BlockSpec((1, D), lambda i, j: (0, 0)),
        pl.BlockSpec((1, D), lambda i, j: (0, 0)),
    ]
    args = [seq, lbl, tok_p, pos, seg, gam, bet]
    aliases = {}
    if carry is not None:
        in_specs.append(pl.BlockSpec(memory_space=pl.ANY))
        args.append(carry)
        aliases = {7: 0}

    return pl.pallas_call(
        body,
        grid=(NSBP, B),
        in_specs=in_specs,
        out_specs=pl.BlockSpec((1, TS, D), lambda i, j: (j, i0 + i, 0)),
        out_shape=jax.ShapeDtypeStruct((B, S, D), jnp.float32),
        input_output_aliases=aliases,
    )(*args)


def kernel(sequence, segment_label, token_table, pos_table, seg_table, gamma,
           beta):
    B, S = sequence.shape
    V, D = token_table.shape
    NSEG = seg_table.shape[0]
    SP = S // SPLIT
    seq = sequence.astype(jnp.int32)
    lbl = segment_label.astype(jnp.int32)
    gam = gamma.reshape(1, D)
    bet = beta.reshape(1, D)
    pos = pos_table[:S]

    sc_gather = _make_sc_gather(B * SP, V, D)
    gathered = [
        sc_gather(lax.slice(seq, (0, p * SP), (B, (p + 1) * SP))
                  .reshape(-1), token_table).reshape(B, SP, D)
        for p in range(SPLIT)
    ]

    out = None
    for p in range(SPLIT):
        out = _tc_ln_piece(B, S, D, NSEG, p, seq, lbl, gathered[p], pos,
                           seg_table, gam, bet, out)
    return out


# speedup vs baseline: 1.1369x; 1.0606x over previous
import functools

import jax
import jax.numpy as jnp
from jax import lax
from jax.experimental import pallas as pl
from jax.experimental.pallas import tpu as pltpu
from jax.experimental.pallas import tpu_sc as plsc

PAD = 0
EPS = 1e-5
NC = 2
NS = 16
NW = NC * NS
SPLIT = 2
TS = 1024
CR = 64


def _make_sc_gather(N, V, D):
    TPW = N // NW
    NCH = max(TPW // CR, 1)
    CRW = min(TPW, CR)

    mesh = plsc.VectorSubcoreMesh(core_axis_name="c", subcore_axis_name="s")

    @functools.partial(
        pl.kernel,
        out_type=jax.ShapeDtypeStruct((N, D), jnp.float32),
        mesh=mesh,
        scratch_types=[
            pltpu.VMEM((TPW,), jnp.int32),
            pltpu.VMEM((CRW, D), jnp.float32),
            pltpu.VMEM((CRW, D), jnp.float32),
            pltpu.SemaphoreType.DMA,
            pltpu.SemaphoreType.DMA,
            pltpu.SemaphoreType.DMA,
            pltpu.SemaphoreType.DMA,
        ],
    )
    def sc_gather(idx_hbm, tab_hbm, out_hbm, idx_v, buf0, buf1,
                  sg0, sg1, so0, so1):
        wid = lax.axis_index("s") * NC + lax.axis_index("c")
        base = wid * TPW
        pltpu.sync_copy(idx_hbm.at[pl.ds(base, TPW)], idx_v)

        bufs = (buf0, buf1)
        sgs = (sg0, sg1)
        sos = (so0, so1)
        gh = {}
        oh = {}
        gh[0] = pltpu.async_copy(
            tab_hbm.at[idx_v.at[pl.ds(0, CRW)]], bufs[0], sgs[0])
        for ci in range(NCH):
            if ci + 1 < NCH:
                if ci - 1 >= 0:
                    oh[ci - 1].wait()
                gh[ci + 1] = pltpu.async_copy(
                    tab_hbm.at[idx_v.at[pl.ds((ci + 1) * CRW, CRW)]],
                    bufs[(ci + 1) % 2], sgs[(ci + 1) % 2])
            gh[ci].wait()
            oh[ci] = pltpu.async_copy(
                bufs[ci % 2], out_hbm.at[pl.ds(base + ci * CRW, CRW)],
                sos[ci % 2])
        for ci in range(max(NCH - 2, 0), NCH):
            oh[ci].wait()

    return sc_gather


def _tc_ln_piece(B, S, D, NSEG, p, seq, lbl, tok_p, pos, seg, gam, bet,
                 carry):
    SP = S // SPLIT
    NSBP = SP // TS
    i0 = p * NSBP

    def body(seq_ref, lbl_ref, tok_ref, pos_ref, seg_ref, gam_ref, bet_ref,
             *rest):
        o_ref = rest[-1]
        b = pl.program_id(1)
        tok = tok_ref[0]
        pos_x = pos_ref[...]
        m = (seq_ref[b] != PAD).astype(jnp.float32)[:, None]
        lbf = lbl_ref[b].astype(jnp.float32)[:, None]
        seg0 = seg_ref[0:1, :]
        dseg = seg_ref[1:2, :] - seg0
        x = (tok + pos_x) * m + seg0 + lbf * dseg
        mean = jnp.mean(x, axis=1, keepdims=True)
        xc = x - mean
        var = jnp.mean(xc * xc, axis=1, keepdims=True)
        o_ref[0] = xc * lax.rsqrt(var + EPS) * gam_ref[...] + bet_ref[...]

    in_specs = [
        pl.BlockSpec((B, TS), lambda i, j: (0, i0 + i)),
        pl.BlockSpec((B, TS), lambda i, j: (0, i0 + i)),
        pl.BlockSpec((1, TS, D), lambda i, j: (j, i, 0)),
        pl.BlockSpec((TS, D), lambda i, j: (i0 + i, 0)),
        pl.BlockSpec((NSEG, D), lambda i, j: (0, 0)),
        pl.BlockSpec((1, D), lambda i, j: (0, 0)),
        pl.BlockSpec((1, D), lambda i, j: (0, 0)),
    ]
    args = [seq, lbl, tok_p, pos, seg, gam, bet]
    aliases = {}
    if carry is not None:
        in_specs.append(pl.BlockSpec(memory_space=pl.ANY))
        args.append(carry)
        aliases = {7: 0}

    return pl.pallas_call(
        body,
        grid=(NSBP, B),
        in_specs=in_specs,
        out_specs=pl.BlockSpec((1, TS, D), lambda i, j: (j, i0 + i, 0)),
        out_shape=jax.ShapeDtypeStruct((B, S, D), jnp.float32),
        input_output_aliases=aliases,
    )(*args)


def kernel(sequence, segment_label, token_table, pos_table, seg_table, gamma,
           beta):
    B, S = sequence.shape
    V, D = token_table.shape
    NSEG = seg_table.shape[0]
    SP = S // SPLIT
    seq = sequence.astype(jnp.int32)
    lbl = segment_label.astype(jnp.int32)
    gam = gamma.reshape(1, D)
    bet = beta.reshape(1, D)
    pos = pos_table[:S]

    sc_gather = _make_sc_gather(B * SP, V, D)
    gathered = [
        sc_gather(lax.slice(seq, (0, p * SP), (B, (p + 1) * SP))
                  .reshape(-1), token_table).reshape(B, SP, D)
        for p in range(SPLIT)
    ]

    out = None
    for p in range(SPLIT):
        out = _tc_ln_piece(B, S, D, NSEG, p, seq, lbl, gathered[p], pos,
                           seg_table, gam, bet, out)
    return out


# speedup vs baseline: 1.1395x; 1.0023x over previous
import functools

import jax
import jax.numpy as jnp
from jax import lax
from jax.experimental import pallas as pl
from jax.experimental.pallas import tpu as pltpu
from jax.experimental.pallas import tpu_sc as plsc

PAD = 0
EPS = 1e-5
NC = 2
NS = 16
NW = NC * NS
SPLIT = 2
TS = 1024
CR = 64


def _make_sc_gather(B, S, SP, p, V, D):
    N = B * SP
    TPW = N // NW
    WPB = NW // B
    NCH = max(TPW // CR, 1)
    CRW = min(TPW, CR)

    mesh = plsc.VectorSubcoreMesh(core_axis_name="c", subcore_axis_name="s")

    @functools.partial(
        pl.kernel,
        out_type=jax.ShapeDtypeStruct((N, D), jnp.float32),
        mesh=mesh,
        scratch_types=[
            pltpu.VMEM((TPW,), jnp.int32),
            pltpu.VMEM((CRW, D), jnp.float32),
            pltpu.VMEM((CRW, D), jnp.float32),
            pltpu.SemaphoreType.DMA,
            pltpu.SemaphoreType.DMA,
            pltpu.SemaphoreType.DMA,
            pltpu.SemaphoreType.DMA,
        ],
    )
    def sc_gather(idx_hbm, tab_hbm, out_hbm, idx_v, buf0, buf1,
                  sg0, sg1, so0, so1):
        wid = lax.axis_index("s") * NC + lax.axis_index("c")
        base = wid * TPW
        gbase = (wid // WPB) * S + p * SP + (wid % WPB) * TPW
        pltpu.sync_copy(idx_hbm.at[pl.ds(gbase, TPW)], idx_v)

        bufs = (buf0, buf1)
        sgs = (sg0, sg1)
        sos = (so0, so1)
        gh = {}
        oh = {}
        gh[0] = pltpu.async_copy(
            tab_hbm.at[idx_v.at[pl.ds(0, CRW)]], bufs[0], sgs[0])
        for ci in range(NCH):
            if ci + 1 < NCH:
                if ci - 1 >= 0:
                    oh[ci - 1].wait()
                gh[ci + 1] = pltpu.async_copy(
                    tab_hbm.at[idx_v.at[pl.ds((ci + 1) * CRW, CRW)]],
                    bufs[(ci + 1) % 2], sgs[(ci + 1) % 2])
            gh[ci].wait()
            oh[ci] = pltpu.async_copy(
                bufs[ci % 2], out_hbm.at[pl.ds(base + ci * CRW, CRW)],
                sos[ci % 2])
        for ci in range(max(NCH - 2, 0), NCH):
            oh[ci].wait()

    return sc_gather


def _tc_ln_piece(B, S, D, NSEG, p, seq, lbl, tok_p, pos, seg, gam, bet,
                 carry):
    SP = S // SPLIT
    NSBP = SP // TS
    i0 = p * NSBP

    def body(seq_ref, lbl_ref, tok_ref, pos_ref, seg_ref, gam_ref, bet_ref,
             *rest):
        o_ref = rest[-1]
        b = pl.program_id(1)
        tok = tok_ref[0]
        pos_x = pos_ref[...]
        m = (seq_ref[b] != PAD).astype(jnp.float32)[:, None]
        lbf = lbl_ref[b].astype(jnp.float32)[:, None]
        seg0 = seg_ref[0:1, :]
        dseg = seg_ref[1:2, :] - seg0
        x = (tok + pos_x) * m + seg0 + lbf * dseg
        mean = jnp.mean(x, axis=1, keepdims=True)
        xc = x - mean
        var = jnp.mean(xc * xc, axis=1, keepdims=True)
        o_ref[0] = xc * lax.rsqrt(var + EPS) * gam_ref[...] + bet_ref[...]

    in_specs = [
        pl.BlockSpec((B, TS), lambda i, j: (0, i0 + i)),
        pl.BlockSpec((B, TS), lambda i, j: (0, i0 + i)),
        pl.BlockSpec((1, TS, D), lambda i, j: (j, i, 0)),
        pl.BlockSpec((TS, D), lambda i, j: (i0 + i, 0)),
        pl.BlockSpec((NSEG, D), lambda i, j: (0, 0)),
        pl.BlockSpec((1, D), lambda i, j: (0, 0)),
        pl.BlockSpec((1, D), lambda i, j: (0, 0)),
    ]
    args = [seq, lbl, tok_p, pos, seg, gam, bet]
    aliases = {}
    if carry is not None:
        in_specs.append(pl.BlockSpec(memory_space=pl.ANY))
        args.append(carry)
        aliases = {7: 0}

    return pl.pallas_call(
        body,
        grid=(NSBP, B),
        in_specs=in_specs,
        out_specs=pl.BlockSpec((1, TS, D), lambda i, j: (j, i0 + i, 0)),
        out_shape=jax.ShapeDtypeStruct((B, S, D), jnp.float32),
        input_output_aliases=aliases,
    )(*args)


def kernel(sequence, segment_label, token_table, pos_table, seg_table, gamma,
           beta):
    B, S = sequence.shape
    V, D = token_table.shape
    NSEG = seg_table.shape[0]
    SP = S // SPLIT
    seq = sequence.astype(jnp.int32)
    lbl = segment_label.astype(jnp.int32)
    gam = gamma.reshape(1, D)
    bet = beta.reshape(1, D)
    pos = pos_table[:S]

    seq_flat = seq.reshape(-1)
    gathered = [
        _make_sc_gather(B, S, SP, p, V, D)(seq_flat, token_table)
        .reshape(B, SP, D)
        for p in range(SPLIT)
    ]

    out = None
    for p in range(SPLIT):
        out = _tc_ln_piece(B, S, D, NSEG, p, seq, lbl, gathered[p], pos,
                           seg_table, gam, bet, out)
    return out
